# baseline (speedup 1.0000x reference)
import functools

import jax
import jax.numpy as jnp
from jax import lax
from jax.experimental import pallas as pl
from jax.experimental.pallas import tpu as pltpu
from jax.experimental.pallas import tpu_sc as plsc

B = 4096
K = 50
M = 100000
D = 128
L = 16
ND = D // L
NW = 32
WIN_B = 4
WIN = WIN_B * K
NSUB = 2
SUBW = WIN // NSUB
BPW = B // NW
CH = BPW // WIN_B
NBUF = 4


def _make_sc_kernel():
    mesh = plsc.VectorSubcoreMesh(
        core_axis_name="core", subcore_axis_name="subcore"
    )

    @functools.partial(
        pl.kernel,
        out_type=jax.ShapeDtypeStruct((K, B, D), jnp.float32),
        mesh=mesh,
        scratch_types=[
            pltpu.VMEM((CH, NSUB, SUBW), jnp.int32),
            pltpu.VMEM((BPW, D), jnp.float32),
            pltpu.VMEM((WIN, D), jnp.float32),
            pltpu.VMEM((WIN, D), jnp.float32),
            pltpu.VMEM((WIN, D), jnp.float32),
            pltpu.VMEM((WIN, D), jnp.float32),
            pltpu.SemaphoreType.DMA,
            pltpu.SemaphoreType.DMA,
            pltpu.SemaphoreType.DMA,
            pltpu.SemaphoreType.DMA,
            pltpu.SemaphoreType.DMA,
            pltpu.SemaphoreType.DMA,
            pltpu.SemaphoreType.DMA,
            pltpu.SemaphoreType.DMA,
        ],
        compiler_params=pltpu.CompilerParams(use_tc_tiling_on_sc=False),
    )
    def run(table_hbm, idx_hbm, pts_hbm, out_hbm,
            idx_v, pts_v, r0, r1, r2, r3,
            gs0, gs1, gs2, gs3, os0, os1, os2, os3):
        rows = (r0, r1, r2, r3)
        gsem = (gs0, gs1, gs2, gs3)
        osem = (os0, os1, os2, os3)

        wid = lax.axis_index("core") * 16 + lax.axis_index("subcore")

        pltpu.sync_copy(idx_hbm.at[pl.ds(wid * CH, CH)], idx_v)
        pltpu.sync_copy(pts_hbm.at[pl.ds(wid * BPW, BPW)], pts_v)

        def start_gather(q, j):
            for s in range(NSUB):
                pltpu.async_copy(
                    table_hbm.at[idx_v.at[q, s]],
                    rows[j].at[pl.ds(s * SUBW, SUBW)],
                    gsem[j],
                )

        def wait_gather(q, j):
            for s in range(NSUB):
                pltpu.make_async_copy(
                    table_hbm.at[idx_v.at[q, s]],
                    rows[j].at[pl.ds(s * SUBW, SUBW)],
                    gsem[j],
                ).wait()

        def wb_pair(q, j, b):
            src = rows[j].at[pl.ds(b * K, K)]
            dst = out_hbm.at[:, wid * BPW + q * WIN_B + b, :]
            return src, dst

        start_gather(0, 0)
        start_gather(1, 1)

        @pl.loop(0, CH, step=NBUF)
        def _(qq):
            for j in range(NBUF):
                q = qq + j

                j2 = (j + 2) % NBUF

                @pl.when(q + 2 < CH)
                def _():
                    @pl.when(q >= 2)
                    def _():
                        for b in range(WIN_B):
                            src, dst = wb_pair(q - 2, j2, b)
                            pltpu.make_async_copy(src, dst, osem[j2]).wait()

                    start_gather(q + 2, j2)

                wait_gather(q, j)

                for b in range(WIN_B):
                    bb = q * WIN_B + b
                    pvecs = [pts_v[bb, pl.ds(d * L, L)] for d in range(ND)]

                    @plsc.parallel_loop(0, K, unroll=2)
                    def _(k, b=b, pvecs=pvecs):
                        r = b * K + k
                        for d in range(ND):
                            sl = pl.ds(d * L, L)
                            rows[j][r, sl] = rows[j][r, sl] - pvecs[d]

                for b in range(WIN_B):
                    src, dst = wb_pair(q, j, b)
                    pltpu.async_copy(src, dst, osem[j])

        for jj in range(NBUF):
            q = CH - NBUF + jj
            for b in range(WIN_B):
                src, dst = wb_pair(q, jj, b)
                pltpu.make_async_copy(src, dst, osem[jj]).wait()

    return run


_sc_kernel = _make_sc_kernel()


def kernel(indices, points, sampleLocs):
    idx_flat = indices.astype(jnp.int32).reshape(B // WIN_B, NSUB, SUBW)
    out_kmajor = _sc_kernel(sampleLocs, idx_flat, points)
    return jnp.transpose(out_kmajor, (1, 0, 2))


# speedup vs baseline: 1.0138x; 1.0138x over previous
import functools

import jax
import jax.numpy as jnp
from jax import lax
from jax.experimental import pallas as pl
from jax.experimental.pallas import tpu as pltpu
from jax.experimental.pallas import tpu_sc as plsc

B = 4096
K = 50
M = 100000
D = 128
L = 16
ND = D // L
NW = 32
WIN_B = 4
WIN = WIN_B * K
NSUB = 1
SUBW = WIN // NSUB
BPW = B // NW
CH = BPW // WIN_B
NBUF = 4


def _make_sc_kernel():
    mesh = plsc.VectorSubcoreMesh(
        core_axis_name="core", subcore_axis_name="subcore"
    )

    @functools.partial(
        pl.kernel,
        out_type=jax.ShapeDtypeStruct((K, B, D), jnp.float32),
        mesh=mesh,
        scratch_types=[
            pltpu.VMEM((CH, NSUB, SUBW), jnp.int32),
            pltpu.VMEM((BPW, D), jnp.float32),
            pltpu.VMEM((WIN, D), jnp.float32),
            pltpu.VMEM((WIN, D), jnp.float32),
            pltpu.VMEM((WIN, D), jnp.float32),
            pltpu.VMEM((WIN, D), jnp.float32),
            pltpu.SemaphoreType.DMA,
            pltpu.SemaphoreType.DMA,
            pltpu.SemaphoreType.DMA,
            pltpu.SemaphoreType.DMA,
            pltpu.SemaphoreType.DMA,
            pltpu.SemaphoreType.DMA,
            pltpu.SemaphoreType.DMA,
            pltpu.SemaphoreType.DMA,
        ],
        compiler_params=pltpu.CompilerParams(use_tc_tiling_on_sc=False),
    )
    def run(table_hbm, idx_hbm, pts_hbm, out_hbm,
            idx_v, pts_v, r0, r1, r2, r3,
            gs0, gs1, gs2, gs3, os0, os1, os2, os3):
        rows = (r0, r1, r2, r3)
        gsem = (gs0, gs1, gs2, gs3)
        osem = (os0, os1, os2, os3)

        wid = lax.axis_index("core") * 16 + lax.axis_index("subcore")

        pltpu.sync_copy(idx_hbm.at[pl.ds(wid * CH, CH)], idx_v)
        pltpu.sync_copy(pts_hbm.at[pl.ds(wid * BPW, BPW)], pts_v)

        def start_gather(q, j):
            for s in range(NSUB):
                pltpu.async_copy(
                    table_hbm.at[idx_v.at[q, s]],
                    rows[j].at[pl.ds(s * SUBW, SUBW)],
                    gsem[j],
                )

        def wait_gather(q, j):
            for s in range(NSUB):
                pltpu.make_async_copy(
                    table_hbm.at[idx_v.at[q, s]],
                    rows[j].at[pl.ds(s * SUBW, SUBW)],
                    gsem[j],
                ).wait()

        def wb_pair(q, j, b):
            src = rows[j].at[pl.ds(b * K, K)]
            dst = out_hbm.at[:, wid * BPW + q * WIN_B + b, :]
            return src, dst

        start_gather(0, 0)
        start_gather(1, 1)

        @pl.loop(0, CH, step=NBUF)
        def _(qq):
            for j in range(NBUF):
                q = qq + j

                j2 = (j + 2) % NBUF

                @pl.when(q + 2 < CH)
                def _():
                    @pl.when(q >= 2)
                    def _():
                        for b in range(WIN_B):
                            src, dst = wb_pair(q - 2, j2, b)
                            pltpu.make_async_copy(src, dst, osem[j2]).wait()

                    start_gather(q + 2, j2)

                wait_gather(q, j)

                for b in range(WIN_B):
                    bb = q * WIN_B + b
                    pvecs = [pts_v[bb, pl.ds(d * L, L)] for d in range(ND)]

                    @plsc.parallel_loop(0, K, unroll=2)
                    def _(k, b=b, pvecs=pvecs):
                        r = b * K + k
                        for d in range(ND):
                            sl = pl.ds(d * L, L)
                            rows[j][r, sl] = rows[j][r, sl] - pvecs[d]

                for b in range(WIN_B):
                    src, dst = wb_pair(q, j, b)
                    pltpu.async_copy(src, dst, osem[j])

        for jj in range(NBUF):
            q = CH - NBUF + jj
            for b in range(WIN_B):
                src, dst = wb_pair(q, jj, b)
                pltpu.make_async_copy(src, dst, osem[jj]).wait()

    return run


_sc_kernel = _make_sc_kernel()


def kernel(indices, points, sampleLocs):
    idx_flat = indices.astype(jnp.int32).reshape(B // WIN_B, NSUB, SUBW)
    out_kmajor = _sc_kernel(sampleLocs, idx_flat, points)
    return jnp.transpose(out_kmajor, (1, 0, 2))
